# baseline (speedup 1.0000x reference)
import functools

import jax
import jax.numpy as jnp
from jax import lax
from jax.experimental import pallas as pl
from jax.experimental.pallas import tpu as pltpu
from jax.experimental.pallas import tpu_sc as plsc

_CPAD = 128


def _sc_gather_y(indexes, labels):
    b = indexes.shape[0]
    nw = 32
    bpw = b // nw
    mesh = plsc.VectorSubcoreMesh(core_axis_name="c", subcore_axis_name="s")

    @functools.partial(
        pl.kernel,
        mesh=mesh,
        out_type=jax.ShapeDtypeStruct((b,), jnp.int32),
        scratch_types=[
            pltpu.VMEM((bpw,), jnp.int32),
            pltpu.VMEM((bpw,), jnp.int32),
            pltpu.SemaphoreType.DMA,
        ],
    )
    def k(idx_hbm, lab_hbm, y_hbm, idx_v, y_v, sem):
        wid = lax.axis_index("s") * 2 + lax.axis_index("c")
        base = wid * bpw
        pltpu.sync_copy(idx_hbm.at[pl.ds(base, bpw)], idx_v)
        pltpu.async_copy(lab_hbm.at[idx_v], y_v, sem).wait()
        pltpu.sync_copy(y_v, y_hbm.at[pl.ds(base, bpw)])

    return k(indexes, labels)


def _nca_tc(xt, labels_row, y_row, idx_row, block_n):
    n, b = xt.shape
    nblk = pl.cdiv(n, block_n)

    def body(xt_ref, lab_ref, y_ref, idx_ref, out_ref, z_acc, t_acc):
        k = pl.program_id(0)
        c_iota = lax.broadcasted_iota(jnp.int32, (_CPAD, 1), 0)

        @pl.when(k == 0)
        def _init():
            z_acc[...] = jnp.zeros_like(z_acc)
            t_acc[...] = jnp.zeros_like(t_acc)

        rows = lax.broadcasted_iota(jnp.int32, (block_n, 1), 0) + k * block_n
        e = jnp.exp(xt_ref[...])
        kill = (rows == idx_ref[...]) | (rows >= n)
        e0 = jnp.where(kill, 0.0, e)
        onehot = (lab_ref[...] == c_iota).astype(jnp.bfloat16)
        z_acc[...] += jnp.sum(e0, axis=0, keepdims=True)
        t_acc[...] += jnp.dot(onehot, e0.astype(jnp.bfloat16),
                              preferred_element_type=jnp.float32)

        @pl.when(k == nblk - 1)
        def _last():
            sel = y_ref[...] == c_iota
            p = jnp.sum(jnp.where(sel, t_acc[...], 0.0), axis=0, keepdims=True)
            prob = p / z_acc[...]
            nz = prob != 0.0
            terms = jnp.where(nz, jnp.log(jnp.where(nz, prob, 1.0)), 0.0)
            out_ref[0, 0] = -jnp.sum(terms) / jnp.float32(b)

    out = pl.pallas_call(
        body,
        grid=(nblk,),
        in_specs=[
            pl.BlockSpec((block_n, b), lambda k: (k, 0)),
            pl.BlockSpec((1, block_n), lambda k: (0, k)),
            pl.BlockSpec((1, b), lambda k: (0, 0)),
            pl.BlockSpec((1, b), lambda k: (0, 0)),
        ],
        out_specs=pl.BlockSpec((1, 1), lambda k: (0, 0),
                               memory_space=pltpu.SMEM),
        out_shape=jax.ShapeDtypeStruct((1, 1), jnp.float32),
        scratch_shapes=[
            pltpu.VMEM((1, b), jnp.float32),
            pltpu.VMEM((_CPAD, b), jnp.float32),
        ],
        compiler_params=pltpu.CompilerParams(
            dimension_semantics=("arbitrary",),
        ),
    )(xt, labels_row, y_row, idx_row)
    return out[0, 0]


def kernel(x, indexes, labels):
    b, n = x.shape
    y = _sc_gather_y(indexes, labels)
    return _nca_tc(x.T, labels.reshape(1, n), y.reshape(1, b),
                   indexes.reshape(1, b), block_n=4096)


# speedup vs baseline: 1.1802x; 1.1802x over previous
import functools

import jax
import jax.numpy as jnp
from jax import lax
from jax.experimental import pallas as pl
from jax.experimental.pallas import tpu as pltpu
from jax.experimental.pallas import tpu_sc as plsc

_CPAD = 128


def _sc_gather(xt, indexes, labels):
    n, b = xt.shape
    nw = 32
    bpw = b // nw
    mesh = plsc.VectorSubcoreMesh(core_axis_name="c", subcore_axis_name="s")

    @functools.partial(
        pl.kernel,
        mesh=mesh,
        out_type=[
            jax.ShapeDtypeStruct((b,), jnp.int32),
            jax.ShapeDtypeStruct((b,), jnp.float32),
        ],
        scratch_types=[
            pltpu.VMEM((bpw,), jnp.int32),
            pltpu.VMEM((bpw,), jnp.int32),
            pltpu.VMEM((bpw, 1024), jnp.float32),
            pltpu.VMEM((bpw,), jnp.float32),
            pltpu.SemaphoreType.DMA,
            pltpu.SemaphoreType.DMA,
        ],
    )
    def k(xt_hbm, idx_hbm, lab_hbm, y_hbm, xe_hbm,
          idx_v, y_v, rows_v, xe_v, sem, sem2):
        wid = lax.axis_index("s") * 2 + lax.axis_index("c")
        base = wid * bpw
        pltpu.sync_copy(idx_hbm.at[pl.ds(base, bpw)], idx_v)
        row_cp = pltpu.async_copy(xt_hbm.at[idx_v], rows_v, sem2)
        pltpu.async_copy(lab_hbm.at[idx_v], y_v, sem).wait()
        pltpu.sync_copy(y_v, y_hbm.at[pl.ds(base, bpw)])
        row_cp.wait()
        lane = lax.iota(jnp.int32, 16)
        for j in range(bpw // 16):
            start = base + j * 16
            acc = jnp.zeros((16,), jnp.float32)
            for t in range(16):
                v = rows_v[j * 16 + t, pl.ds(start, 16)]
                acc = jnp.where(lane == t, v, acc)
            xe_v[pl.ds(j * 16, 16)] = acc
        pltpu.sync_copy(xe_v, xe_hbm.at[pl.ds(base, bpw)])

    return k(xt, indexes, labels)


def _nca_tc(xt, labels_row, y_row, xe_row, block_n):
    n, b = xt.shape
    nblk = pl.cdiv(n, block_n)
    tail = n - (nblk - 1) * block_n

    def body(xt_ref, lab_ref, y_ref, xe_ref, out_ref, z_acc, t_acc):
        k = pl.program_id(0)
        c_iota = lax.broadcasted_iota(jnp.int32, (_CPAD, 1), 0)

        @pl.when(k == 0)
        def _init():
            z_acc[...] = jnp.zeros_like(z_acc)
            t_acc[...] = jnp.zeros_like(t_acc)

        e = jnp.exp(xt_ref[...])
        onehot = (lab_ref[...] == c_iota).astype(jnp.bfloat16)

        @pl.when(k < nblk - 1)
        def _full():
            z_acc[...] += jnp.sum(e, axis=0, keepdims=True)
            t_acc[...] += jnp.dot(onehot, e.astype(jnp.bfloat16),
                                  preferred_element_type=jnp.float32)

        @pl.when(k == nblk - 1)
        def _last():
            rows = lax.broadcasted_iota(jnp.int32, (block_n, 1), 0)
            e0 = jnp.where(rows >= tail, 0.0, e)
            z = z_acc[...] + jnp.sum(e0, axis=0, keepdims=True)
            t = t_acc[...] + jnp.dot(onehot, e0.astype(jnp.bfloat16),
                                     preferred_element_type=jnp.float32)
            es = jnp.exp(xe_ref[...])
            esb = es.astype(jnp.bfloat16).astype(jnp.float32)
            sel = y_ref[...] == c_iota
            p = jnp.sum(jnp.where(sel, t, 0.0), axis=0, keepdims=True) - esb
            z = z - es
            prob = p / z
            nz = p != 0.0
            terms = jnp.where(nz, jnp.log(jnp.where(nz, prob, 1.0)), 0.0)
            out_ref[0, 0] = -jnp.sum(terms) / jnp.float32(b)

    out = pl.pallas_call(
        body,
        grid=(nblk,),
        in_specs=[
            pl.BlockSpec((block_n, b), lambda k: (k, 0)),
            pl.BlockSpec((1, block_n), lambda k: (0, k)),
            pl.BlockSpec((1, b), lambda k: (0, 0)),
            pl.BlockSpec((1, b), lambda k: (0, 0)),
        ],
        out_specs=pl.BlockSpec((1, 1), lambda k: (0, 0),
                               memory_space=pltpu.SMEM),
        out_shape=jax.ShapeDtypeStruct((1, 1), jnp.float32),
        scratch_shapes=[
            pltpu.VMEM((1, b), jnp.float32),
            pltpu.VMEM((_CPAD, b), jnp.float32),
        ],
        compiler_params=pltpu.CompilerParams(
            dimension_semantics=("arbitrary",),
        ),
    )(xt, labels_row, y_row, xe_row)
    return out[0, 0]


def kernel(x, indexes, labels):
    b, n = x.shape
    xt = x.T
    y, xe = _sc_gather(xt, indexes, labels)
    return _nca_tc(xt, labels.reshape(1, n), y.reshape(1, b),
                   xe.reshape(1, b), block_n=2048)


# speedup vs baseline: 1.2443x; 1.0543x over previous
import functools

import jax
import jax.numpy as jnp
from jax import lax
from jax.experimental import pallas as pl
from jax.experimental.pallas import tpu as pltpu
from jax.experimental.pallas import tpu_sc as plsc

_CPAD = 128


def _sc_gather(xt, indexes, labels):
    n, b = xt.shape
    nw = 32
    bpw = b // nw
    mesh = plsc.VectorSubcoreMesh(core_axis_name="c", subcore_axis_name="s")

    @functools.partial(
        pl.kernel,
        mesh=mesh,
        out_type=[
            jax.ShapeDtypeStruct((b,), jnp.int32),
            jax.ShapeDtypeStruct((b,), jnp.float32),
        ],
        scratch_types=[
            pltpu.VMEM((bpw,), jnp.int32),
            pltpu.VMEM((bpw,), jnp.int32),
            pltpu.VMEM((bpw, 1024), jnp.float32),
            pltpu.VMEM((bpw,), jnp.float32),
            pltpu.SemaphoreType.DMA,
            pltpu.SemaphoreType.DMA,
        ],
    )
    def k(xt_hbm, idx_hbm, lab_hbm, y_hbm, xe_hbm,
          idx_v, y_v, rows_v, xe_v, sem, sem2):
        wid = lax.axis_index("s") * 2 + lax.axis_index("c")
        base = wid * bpw
        pltpu.sync_copy(idx_hbm.at[pl.ds(base, bpw)], idx_v)
        row_cp = pltpu.async_copy(xt_hbm.at[idx_v], rows_v, sem2)
        pltpu.async_copy(lab_hbm.at[idx_v], y_v, sem).wait()
        pltpu.sync_copy(y_v, y_hbm.at[pl.ds(base, bpw)])
        row_cp.wait()
        lane = lax.iota(jnp.int32, 16)
        for j in range(bpw // 16):
            start = base + j * 16
            acc = jnp.zeros((16,), jnp.float32)
            for t in range(16):
                v = rows_v[j * 16 + t, pl.ds(start, 16)]
                acc = jnp.where(lane == t, v, acc)
            xe_v[pl.ds(j * 16, 16)] = acc
        pltpu.sync_copy(xe_v, xe_hbm.at[pl.ds(base, bpw)])

    return k(xt, indexes, labels)


def _nca_tc(xt, labels_row, y_row, xe_row, block_n):
    n, b = xt.shape
    nblk = pl.cdiv(n, block_n)
    tail = n - (nblk - 1) * block_n

    def body(xt_ref, lab_ref, y_ref, xe_ref, out_ref, t_acc):
        k = pl.program_id(0)
        c_iota = lax.broadcasted_iota(jnp.int32, (_CPAD, 1), 0)

        @pl.when(k == 0)
        def _init():
            t_acc[...] = jnp.zeros_like(t_acc)

        e = jnp.exp(xt_ref[...])
        onehot = ((lab_ref[...] == c_iota) |
                  (c_iota == _CPAD - 1)).astype(jnp.bfloat16)

        @pl.when(k < nblk - 1)
        def _full():
            t_acc[...] += jnp.dot(onehot, e.astype(jnp.bfloat16),
                                  preferred_element_type=jnp.float32)

        @pl.when(k == nblk - 1)
        def _last():
            rows = lax.broadcasted_iota(jnp.int32, (block_n, 1), 0)
            e0 = jnp.where(rows >= tail, 0.0, e)
            t = t_acc[...] + jnp.dot(onehot, e0.astype(jnp.bfloat16),
                                     preferred_element_type=jnp.float32)
            es = jnp.exp(xe_ref[...])
            esb = es.astype(jnp.bfloat16).astype(jnp.float32)
            sel = y_ref[...] == c_iota
            p = jnp.sum(jnp.where(sel, t, 0.0), axis=0, keepdims=True) - esb
            z = t[_CPAD - 1:_CPAD, :] - esb
            prob = p / z
            nz = p != 0.0
            terms = jnp.where(nz, jnp.log(jnp.where(nz, prob, 1.0)), 0.0)
            out_ref[0, 0] = -jnp.sum(terms) / jnp.float32(b)

    out = pl.pallas_call(
        body,
        grid=(nblk,),
        in_specs=[
            pl.BlockSpec((block_n, b), lambda k: (k, 0)),
            pl.BlockSpec((1, block_n), lambda k: (0, k)),
            pl.BlockSpec((1, b), lambda k: (0, 0)),
            pl.BlockSpec((1, b), lambda k: (0, 0)),
        ],
        out_specs=pl.BlockSpec((1, 1), lambda k: (0, 0),
                               memory_space=pltpu.SMEM),
        out_shape=jax.ShapeDtypeStruct((1, 1), jnp.float32),
        scratch_shapes=[
            pltpu.VMEM((_CPAD, b), jnp.float32),
        ],
        compiler_params=pltpu.CompilerParams(
            dimension_semantics=("arbitrary",),
        ),
    )(xt, labels_row, y_row, xe_row)
    return out[0, 0]


def kernel(x, indexes, labels):
    b, n = x.shape
    xt = x.T
    y, xe = _sc_gather(xt, indexes, labels)
    return _nca_tc(xt, labels.reshape(1, n), y.reshape(1, b),
                   xe.reshape(1, b), block_n=2048)
